# baseline (speedup 1.0000x reference)
import functools

import jax
import jax.numpy as jnp
from jax import lax
from jax.experimental import pallas as pl
from jax.experimental.pallas import tpu as pltpu
from jax.experimental.pallas import tpu_sc as plsc

N = 10000
E = 160000
F = 128
HID = 256
C = 40
DP = 48
DEGC = C
NLAYERS = 2
TEMP = 0.5

NC, NS = 2, 16
NW = NC * NS
EC = 128
EPAD = 163840
TPT = EPAD // NW // EC
NPAD = 10112
RPT = NPAD // NS
BR = 1000
SEG = 1600


def _deg_terms(deg):
    pos = deg > 0.0
    dmax = jnp.maximum(deg, 1e-12)
    dis = jnp.where(pos, lax.rsqrt(dmax), 0.0)
    dinv = jnp.where(pos, 1.0 / dmax, 0.0)
    inv1 = 1.0 / jnp.maximum(deg, 1.0)
    return dis, dinv, inv1


def _post_scale(deg, sym, mean):
    dis, dinv, inv1 = _deg_terms(deg)
    one = jnp.ones_like(deg)
    return jnp.where(sym, dis, dinv) * jnp.where(mean, inv1, one)


def _sc_mesh():
    return plsc.VectorSubcoreMesh(core_axis_name="c", subcore_axis_name="s",
                                  num_cores=NC, num_subcores=NS)


def _sc_prop_sum(h_pad, src2d, dst2d, zrows):
    nbuf = 8
    nrounds = TPT // nbuf

    @functools.partial(
        pl.kernel,
        out_type=jax.ShapeDtypeStruct((NC, NPAD, DP), jnp.float32),
        mesh=_sc_mesh(),
        scratch_types=[
            pltpu.VMEM((TPT, EC), jnp.int32),
            pltpu.VMEM((TPT, EC), jnp.int32),
            [pltpu.VMEM((EC, DP), jnp.float32)] * nbuf,
            [pltpu.SemaphoreType.DMA] * nbuf,
            [pltpu.SemaphoreType.DMA] * nbuf,
            pltpu.VMEM_SHARED((NPAD, DP), jnp.float32),
            pltpu.VMEM_SHARED((N, DP), jnp.float32),
        ],
        compiler_params=pltpu.CompilerParams(use_tc_tiling_on_sc=False),
    )
    def kfn(h_hbm, s_hbm, d_hbm, z_hbm, out_hbm, sidx, didx, rows, gsem,
            ssem, acc, hsp):
        c = lax.axis_index("c")
        s = lax.axis_index("s")
        pltpu.sync_copy(z_hbm, acc.at[pl.ds(s * RPT, RPT)])

        @pl.when(s == 0)
        def _():
            pltpu.sync_copy(h_hbm, hsp)

        tb = (c * NS + s) * TPT
        pltpu.sync_copy(s_hbm.at[pl.ds(tb, TPT)], sidx)
        pltpu.sync_copy(d_hbm.at[pl.ds(tb, TPT)], didx)
        plsc.subcore_barrier()

        def gstart(j, b):
            pltpu.async_copy(hsp.at[sidx.at[j]], rows[b], gsem[b])

        def gwait(j, b):
            pltpu.make_async_copy(hsp.at[sidx.at[j]], rows[b],
                                  gsem[b]).wait()

        def sstart(j, b):
            pltpu.async_copy(rows[b], acc.at[didx.at[j]], ssem[b], add=True)

        def swait(j, b):
            pltpu.make_async_copy(rows[b], acc.at[didx.at[j]],
                                  ssem[b]).wait()

        for b in range(nbuf):
            gstart(b, b)

        def body(i, carry):
            j0 = i * nbuf
            for b in range(nbuf):
                gwait(j0 + b, b)
                sstart(j0 + b, b)
            for b in range(nbuf):
                swait(j0 + b, b)

                @pl.when(i < nrounds - 1)
                def _(jb=j0 + b + nbuf, bb=b):
                    gstart(jb, bb)
            return carry

        lax.fori_loop(0, nrounds, body, 0)
        plsc.subcore_barrier()
        pltpu.sync_copy(acc.at[pl.ds(s * RPT, RPT)],
                        out_hbm.at[c, pl.ds(s * RPT, RPT)])

    return kfn(h_pad, src2d, dst2d, zrows)


def _sc_degree(dst2d, orows, z16):
    @functools.partial(
        pl.kernel,
        out_type=jax.ShapeDtypeStruct((NC, NPAD, 16), jnp.float32),
        mesh=_sc_mesh(),
        scratch_types=[
            pltpu.VMEM((TPT, EC), jnp.int32),
            pltpu.VMEM((EC, 16), jnp.float32),
            pltpu.VMEM_SHARED((NPAD, 16), jnp.float32),
        ],
        compiler_params=pltpu.CompilerParams(use_tc_tiling_on_sc=False),
    )
    def kfn(d_hbm, o_hbm, z_hbm, out_hbm, didx, ones, acc):
        c = lax.axis_index("c")
        s = lax.axis_index("s")
        pltpu.sync_copy(z_hbm, acc.at[pl.ds(s * RPT, RPT)])
        pltpu.sync_copy(o_hbm, ones)
        tb = (c * NS + s) * TPT
        pltpu.sync_copy(d_hbm.at[pl.ds(tb, TPT)], didx)
        plsc.subcore_barrier()

        def body(j, carry):
            pltpu.sync_copy(ones, acc.at[didx.at[j]], add=True)
            return carry

        lax.fori_loop(0, TPT, body, 0)
        plsc.subcore_barrier()
        pltpu.sync_copy(acc.at[pl.ds(s * RPT, RPT)],
                        out_hbm.at[c, pl.ds(s * RPT, RPT)])

    return kfn(dst2d, orows, z16)


def _aug(x40):
    n = x40.shape[0]
    return jnp.concatenate(
        [x40, jnp.ones((n, 1), jnp.float32),
         jnp.zeros((n, DP - C - 1), jnp.float32)], axis=1)


def _tc_premlp(x, W1, b1, W2, b2):
    def body(x_ref, w1_ref, b1_ref, w2_ref, b2_ref, h_ref, hs_ref):
        a = jnp.maximum(
            jnp.dot(x_ref[...], w1_ref[...],
                    preferred_element_type=jnp.float32) + b1_ref[...], 0.0)
        h = jnp.dot(a, w2_ref[...],
                    preferred_element_type=jnp.float32) + b2_ref[...]
        h_ref[...] = h
        hs_ref[...] = _aug(h)

    return pl.pallas_call(
        body,
        grid=(N // BR,),
        in_specs=[
            pl.BlockSpec((BR, F), lambda i: (i, 0)),
            pl.BlockSpec((F, HID), lambda i: (0, 0)),
            pl.BlockSpec((1, HID), lambda i: (0, 0)),
            pl.BlockSpec((HID, C), lambda i: (0, 0)),
            pl.BlockSpec((1, C), lambda i: (0, 0)),
        ],
        out_specs=[
            pl.BlockSpec((BR, C), lambda i: (i, 0)),
            pl.BlockSpec((BR, DP), lambda i: (i, 0)),
        ],
        out_shape=[
            jax.ShapeDtypeStruct((N, C), jnp.float32),
            jax.ShapeDtypeStruct((N, DP), jnp.float32),
        ],
    )(x, W1, b1.reshape(1, HID), W2, b2.reshape(1, C))


def _tc_prep(degp, h):
    def body(dp_ref, h_ref, hs_ref):
        deg = dp_ref[0, :, 0:1] + dp_ref[1, :, 0:1]
        dis, _, _ = _deg_terms(deg)
        hs_ref[...] = _aug(h_ref[...] * dis)

    return pl.pallas_call(
        body,
        grid=(N // BR,),
        in_specs=[
            pl.BlockSpec((NC, BR, 16), lambda i: (0, i, 0)),
            pl.BlockSpec((BR, C), lambda i: (i, 0)),
        ],
        out_specs=pl.BlockSpec((BR, DP), lambda i: (i, 0)),
        out_shape=jax.ShapeDtypeStruct((N, DP), jnp.float32),
    )(degp, h)


def _tc_segmax(hs, src2, dst2):
    def body(src_ref, dst_ref, hs_ref, o_ref):
        @pl.when(pl.program_id(0) == 0)
        def _():
            o_ref[...] = jnp.full((N, DP), -jnp.inf, jnp.float32)

        def step(e, carry):
            sv = src_ref[0, 0, e]
            dv = dst_ref[0, 0, e]
            row = hs_ref[pl.ds(sv, 1), :]
            o_ref[pl.ds(dv, 1), :] = jnp.maximum(o_ref[pl.ds(dv, 1), :], row)
            return carry

        lax.fori_loop(0, SEG, step, 0)

    return pl.pallas_call(
        body,
        grid=(E // SEG,),
        in_specs=[
            pl.BlockSpec((1, 1, SEG), lambda i: (i, 0, 0),
                         memory_space=pltpu.SMEM),
            pl.BlockSpec((1, 1, SEG), lambda i: (i, 0, 0),
                         memory_space=pltpu.SMEM),
            pl.BlockSpec((N, DP), lambda i: (0, 0)),
        ],
        out_specs=pl.BlockSpec((N, DP), lambda i: (0, 0)),
        out_shape=jax.ShapeDtypeStruct((N, DP), jnp.float32),
    )(src2, dst2, hs)


def _tc_mid(p_in, degp, wv, is_max):
    def body(wv_ref, p_ref, dp_ref, o_ref):
        sym = wv_ref[0] > 0.5
        mean = wv_ref[1] > 0.5
        if is_max:
            p48 = p_ref[...]
            p48 = jnp.where(jnp.isfinite(p48), p48, 0.0)
            deg = dp_ref[0, :, 0:1] + dp_ref[1, :, 0:1]
        else:
            p48 = p_ref[0] + p_ref[1]
            deg = p48[:, DEGC:DEGC + 1]
        dis, _, _ = _deg_terms(deg)
        pre = jnp.where(sym, dis, jnp.ones_like(deg))
        o_ref[...] = _aug(p48[:, :C] * (_post_scale(deg, sym, mean) * pre))

    p_spec = (pl.BlockSpec((BR, DP), lambda i: (i, 0)) if is_max
              else pl.BlockSpec((NC, BR, DP), lambda i: (0, i, 0)))
    return pl.pallas_call(
        body,
        grid=(N // BR,),
        in_specs=[
            pl.BlockSpec(memory_space=pltpu.SMEM),
            p_spec,
            pl.BlockSpec((NC, BR, 16), lambda i: (0, i, 0)),
        ],
        out_specs=pl.BlockSpec((BR, DP), lambda i: (i, 0)),
        out_shape=jax.ShapeDtypeStruct((N, DP), jnp.float32),
    )(wv, p_in, degp)


def _tc_combine0(p_in, degp, xprev, Wl, bl, wv, is_max):
    def body(wv_ref, p_ref, dp_ref, xp_ref, w_ref, b_ref, xn_ref, hs_ref):
        sym0 = wv_ref[3] > 0.5
        mean0 = wv_ref[4] > 0.5
        sym1 = wv_ref[5] > 0.5
        if is_max:
            p48 = p_ref[...]
            p48 = jnp.where(jnp.isfinite(p48), p48, 0.0)
            deg = dp_ref[0, :, 0:1] + dp_ref[1, :, 0:1]
        else:
            p48 = p_ref[0] + p_ref[1]
            deg = p48[:, DEGC:DEGC + 1]
        p = p48[:, :C] * _post_scale(deg, sym0, mean0)
        m = jnp.maximum(wv_ref[0] * p, 0.0)
        xp = xp_ref[...]
        cadd = m + xp
        ccat = (jnp.dot(m, w_ref[0:C, :], preferred_element_type=jnp.float32)
                + jnp.dot(xp, w_ref[C:, :],
                          preferred_element_type=jnp.float32)
                + b_ref[...])
        xn = wv_ref[1] * cadd + wv_ref[2] * ccat
        xn_ref[...] = xn
        dis, _, _ = _deg_terms(deg)
        pre1 = jnp.where(sym1, dis, jnp.ones_like(deg))
        hs_ref[...] = _aug(xn * pre1)

    p_spec = (pl.BlockSpec((BR, DP), lambda i: (i, 0)) if is_max
              else pl.BlockSpec((NC, BR, DP), lambda i: (0, i, 0)))
    return pl.pallas_call(
        body,
        grid=(N // BR,),
        in_specs=[
            pl.BlockSpec(memory_space=pltpu.SMEM),
            p_spec,
            pl.BlockSpec((NC, BR, 16), lambda i: (0, i, 0)),
            pl.BlockSpec((BR, C), lambda i: (i, 0)),
            pl.BlockSpec((2 * C, C), lambda i: (0, 0)),
            pl.BlockSpec((1, C), lambda i: (0, 0)),
        ],
        out_specs=[
            pl.BlockSpec((BR, C), lambda i: (i, 0)),
            pl.BlockSpec((BR, DP), lambda i: (i, 0)),
        ],
        out_shape=[
            jax.ShapeDtypeStruct((N, C), jnp.float32),
            jax.ShapeDtypeStruct((N, DP), jnp.float32),
        ],
    )(wv, p_in, degp, xprev, Wl, bl.reshape(1, C))


def _tc_comb1jk(p_in, degp, x1, h0, Wl, bl, jk_W, jk_b, wv, jw, is_max):
    def body(wv_ref, jw_ref, p_ref, dp_ref, x1_ref, h_ref, w_ref, b_ref,
             jw_w_ref, jw_b_ref, o_ref):
        sym1 = wv_ref[3] > 0.5
        mean1 = wv_ref[4] > 0.5
        if is_max:
            p48 = p_ref[...]
            p48 = jnp.where(jnp.isfinite(p48), p48, 0.0)
            deg = dp_ref[0, :, 0:1] + dp_ref[1, :, 0:1]
        else:
            p48 = p_ref[0] + p_ref[1]
            deg = p48[:, DEGC:DEGC + 1]
        p = p48[:, :C] * _post_scale(deg, sym1, mean1)
        m = jnp.maximum(wv_ref[0] * p, 0.0)
        x1 = x1_ref[...]
        cadd = m + x1
        ccat = (jnp.dot(m, w_ref[0:C, :], preferred_element_type=jnp.float32)
                + jnp.dot(x1, w_ref[C:, :],
                          preferred_element_type=jnp.float32)
                + b_ref[...])
        x2 = wv_ref[1] * cadd + wv_ref[2] * ccat
        h = h_ref[...]
        cat = (jnp.dot(h, jw_w_ref[0:C, :], preferred_element_type=jnp.float32)
               + jnp.dot(x1, jw_w_ref[C:2 * C, :],
                         preferred_element_type=jnp.float32)
               + jnp.dot(x2, jw_w_ref[2 * C:, :],
                         preferred_element_type=jnp.float32)
               + jw_b_ref[...])
        mx = jnp.maximum(jnp.maximum(h, x1), x2)
        mn = (h + x1 + x2) / 3.0
        lin = (jw_ref[0] * x2 + jw_ref[1] * mx + jw_ref[2] * mn
               + jw_ref[3] * cat)
        rmax = jnp.max(lin, axis=1, keepdims=True)
        sh = lin - rmax
        o_ref[...] = sh - jnp.log(jnp.sum(jnp.exp(sh), axis=1, keepdims=True))

    p_spec = (pl.BlockSpec((BR, DP), lambda i: (i, 0)) if is_max
              else pl.BlockSpec((NC, BR, DP), lambda i: (0, i, 0)))
    return pl.pallas_call(
        body,
        grid=(N // BR,),
        in_specs=[
            pl.BlockSpec(memory_space=pltpu.SMEM),
            pl.BlockSpec(memory_space=pltpu.SMEM),
            p_spec,
            pl.BlockSpec((NC, BR, 16), lambda i: (0, i, 0)),
            pl.BlockSpec((BR, C), lambda i: (i, 0)),
            pl.BlockSpec((BR, C), lambda i: (i, 0)),
            pl.BlockSpec((2 * C, C), lambda i: (0, 0)),
            pl.BlockSpec((1, C), lambda i: (0, 0)),
            pl.BlockSpec((3 * C, C), lambda i: (0, 0)),
            pl.BlockSpec((1, C), lambda i: (0, 0)),
        ],
        out_specs=pl.BlockSpec((BR, C), lambda i: (i, 0)),
        out_shape=jax.ShapeDtypeStruct((N, C), jnp.float32),
    )(wv, jw, p_in, degp, x1, h0, Wl, bl.reshape(1, C), jk_W,
      jk_b.reshape(1, C))


def _categ(alphas, u):
    ws = jax.nn.softmax((alphas - jnp.log(-jnp.log(u))) / TEMP, axis=-1)
    oh = jax.nn.one_hot(jnp.argmax(ws, axis=-1), ws.shape[-1], dtype=ws.dtype)
    return (oh - ws) + ws


def _gumbels():
    gk = jax.random.key(42)
    g = jax.random.split(gk, 5)
    lo, hi = 1e-6, 1.0 - 1e-6
    return (jax.random.uniform(g[0], (NLAYERS, 2), minval=lo, maxval=hi),
            jax.random.uniform(g[1], (NLAYERS, 3), minval=lo, maxval=hi),
            jax.random.uniform(g[2], (NLAYERS, 2), minval=lo, maxval=hi),
            jax.random.uniform(g[3], (NLAYERS, 2), minval=lo, maxval=hi),
            jax.random.uniform(g[4], (1, 4), minval=lo, maxval=hi))


def kernel(x, edge_index, pre_W1, pre_b1, pre_W2, pre_b2, comb_W, comb_b,
           jk_W, jk_b, neigh_alphas, aggr_alphas, norm_alphas, comb_alphas,
           jk_alphas):
    us = _gumbels()
    nw = _categ(neigh_alphas, us[0])
    aw = _categ(aggr_alphas, us[1])
    sw = _categ(norm_alphas, us[2])
    cw = _categ(comb_alphas, us[3])
    jw = _categ(jk_alphas, us[4])

    src = edge_index[0]
    dst = edge_index[1]
    padn = EPAD // NW - E // NW
    pad_src = jnp.zeros((NW, padn), jnp.int32)
    pad_dst = jnp.broadcast_to(
        N + (jnp.arange(padn, dtype=jnp.int32) % (NPAD - N)), (NW, padn))
    src2d = jnp.concatenate(
        [src.reshape(NW, E // NW), pad_src], axis=1).reshape(EPAD // EC, EC)
    dst2d = jnp.concatenate(
        [dst.reshape(NW, E // NW), pad_dst], axis=1).reshape(EPAD // EC, EC)
    srcseg = src.reshape(E // SEG, 1, SEG)
    dstseg = dst.reshape(E // SEG, 1, SEG)
    zrows = jnp.zeros((RPT, DP), jnp.float32)
    z16 = jnp.zeros((RPT, 16), jnp.float32)
    orows = jnp.ones((EC, 16), jnp.float32)
    zdegp = jnp.zeros((NC, NPAD, 16), jnp.float32)

    h0, hs0 = _tc_premlp(x, pre_W1, pre_b1, pre_W2, pre_b2)

    mean0 = (aw[0, 1] > 0.5).astype(jnp.float32)
    mean1 = (aw[1, 1] > 0.5).astype(jnp.float32)
    sym0 = (sw[0, 0] > 0.5).astype(jnp.float32)
    sym1 = (sw[1, 0] > 0.5).astype(jnp.float32)
    wprod0 = (jnp.sum(nw[0]) * jnp.sum(aw[0])) * jnp.sum(sw[0])
    wprod1 = (jnp.sum(nw[1]) * jnp.sum(aw[1])) * jnp.sum(sw[1])
    wv0 = jnp.stack([wprod0, cw[0, 0], cw[0, 1], sym0, mean0, sym1])
    wv1 = jnp.stack([wprod1, cw[1, 0], cw[1, 1], sym1, mean1])
    wvm0 = jnp.stack([sym0, mean0])
    wvm1 = jnp.stack([sym1, mean1])
    jwv = jw[0]


    def l0_rw_1h_sum():
        parts = _sc_prop_sum(hs0, src2d, dst2d, zrows)
        return _tc_combine0(parts, zdegp, h0, comb_W[0], comb_b[0], wv0,
                            False)

    def l0_rw_1h_max():
        degp = _sc_degree(dst2d, orows, z16)
        pm = _tc_segmax(hs0, srcseg, dstseg)
        return _tc_combine0(pm, degp, h0, comb_W[0], comb_b[0], wv0, True)

    def l0_rw_2h_sum():
        p1 = _sc_prop_sum(hs0, src2d, dst2d, zrows)
        hmid = _tc_mid(p1, zdegp, wvm0, False)
        p2 = _sc_prop_sum(hmid, src2d, dst2d, zrows)
        return _tc_combine0(p2, zdegp, h0, comb_W[0], comb_b[0], wv0, False)

    def l0_rw_2h_max():
        degp = _sc_degree(dst2d, orows, z16)
        pm1 = _tc_segmax(hs0, srcseg, dstseg)
        hmid = _tc_mid(pm1, degp, wvm0, True)
        pm2 = _tc_segmax(hmid, srcseg, dstseg)
        return _tc_combine0(pm2, degp, h0, comb_W[0], comb_b[0], wv0, True)

    def l0_sym_1h_sum():
        degp = _sc_degree(dst2d, orows, z16)
        hss = _tc_prep(degp, h0)
        parts = _sc_prop_sum(hss, src2d, dst2d, zrows)
        return _tc_combine0(parts, degp, h0, comb_W[0], comb_b[0], wv0,
                            False)

    def l0_sym_1h_max():
        degp = _sc_degree(dst2d, orows, z16)
        hss = _tc_prep(degp, h0)
        pm = _tc_segmax(hss, srcseg, dstseg)
        return _tc_combine0(pm, degp, h0, comb_W[0], comb_b[0], wv0, True)

    def l0_sym_2h_sum():
        degp = _sc_degree(dst2d, orows, z16)
        hss = _tc_prep(degp, h0)
        p1 = _sc_prop_sum(hss, src2d, dst2d, zrows)
        hmid = _tc_mid(p1, degp, wvm0, False)
        p2 = _sc_prop_sum(hmid, src2d, dst2d, zrows)
        return _tc_combine0(p2, degp, h0, comb_W[0], comb_b[0], wv0, False)

    def l0_sym_2h_max():
        degp = _sc_degree(dst2d, orows, z16)
        hss = _tc_prep(degp, h0)
        pm1 = _tc_segmax(hss, srcseg, dstseg)
        hmid = _tc_mid(pm1, degp, wvm0, True)
        pm2 = _tc_segmax(hmid, srcseg, dstseg)
        return _tc_combine0(pm2, degp, h0, comb_W[0], comb_b[0], wv0, True)

    bi0 = (4 * (sw[0, 0] > 0.5).astype(jnp.int32)
           + 2 * (nw[0, 1] > 0.5).astype(jnp.int32)
           + (aw[0, 2] > 0.5).astype(jnp.int32))
    x1, hs1 = lax.switch(bi0, [
        l0_rw_1h_sum, l0_rw_1h_max, l0_rw_2h_sum, l0_rw_2h_max,
        l0_sym_1h_sum, l0_sym_1h_max, l0_sym_2h_sum, l0_sym_2h_max,
    ])


    def l1_1h_sum():
        parts = _sc_prop_sum(hs1, src2d, dst2d, zrows)
        return _tc_comb1jk(parts, zdegp, x1, h0, comb_W[1], comb_b[1],
                           jk_W, jk_b, wv1, jwv, False)

    def l1_1h_max():
        degp = _sc_degree(dst2d, orows, z16)
        pm = _tc_segmax(hs1, srcseg, dstseg)
        return _tc_comb1jk(pm, degp, x1, h0, comb_W[1], comb_b[1],
                           jk_W, jk_b, wv1, jwv, True)

    def l1_2h_sum():
        p1 = _sc_prop_sum(hs1, src2d, dst2d, zrows)
        hmid = _tc_mid(p1, zdegp, wvm1, False)
        p2 = _sc_prop_sum(hmid, src2d, dst2d, zrows)
        return _tc_comb1jk(p2, zdegp, x1, h0, comb_W[1], comb_b[1],
                           jk_W, jk_b, wv1, jwv, False)

    def l1_2h_max():
        degp = _sc_degree(dst2d, orows, z16)
        pm1 = _tc_segmax(hs1, srcseg, dstseg)
        hmid = _tc_mid(pm1, degp, wvm1, True)
        pm2 = _tc_segmax(hmid, srcseg, dstseg)
        return _tc_comb1jk(pm2, degp, x1, h0, comb_W[1], comb_b[1],
                           jk_W, jk_b, wv1, jwv, True)

    bi1 = (2 * (nw[1, 1] > 0.5).astype(jnp.int32)
           + (aw[1, 2] > 0.5).astype(jnp.int32))
    return lax.switch(bi1, [l1_1h_sum, l1_1h_max, l1_2h_sum, l1_2h_max])


# speedup vs baseline: 1.1228x; 1.1228x over previous
import functools

import jax
import jax.numpy as jnp
from jax import lax
from jax.experimental import pallas as pl
from jax.experimental.pallas import tpu as pltpu
from jax.experimental.pallas import tpu_sc as plsc

N = 10000
E = 160000
F = 128
HID = 256
C = 40
DP = 48
DEGC = C
NLAYERS = 2
TEMP = 0.5

NC, NS = 2, 16
NW = NC * NS
EC = 128
ETR = E // EC
TPT = 40
NPAD = 10112
RPT = NPAD // NS
BR = 1000
SEG = 1600
XTR = ETR - 39 * NW


def _deg_terms(deg):
    pos = deg > 0.0
    dmax = jnp.maximum(deg, 1e-12)
    dis = jnp.where(pos, lax.rsqrt(dmax), 0.0)
    dinv = jnp.where(pos, 1.0 / dmax, 0.0)
    inv1 = 1.0 / jnp.maximum(deg, 1.0)
    return dis, dinv, inv1


def _post_scale(deg, sym, mean):
    dis, dinv, inv1 = _deg_terms(deg)
    one = jnp.ones_like(deg)
    return jnp.where(sym, dis, dinv) * jnp.where(mean, inv1, one)


def _sc_mesh():
    return plsc.VectorSubcoreMesh(core_axis_name="c", subcore_axis_name="s",
                                  num_cores=NC, num_subcores=NS)


def _sc_prop_sum(h_pad, src2d, dst2d, zrows):
    nbuf = 8
    nrounds = TPT // nbuf

    @functools.partial(
        pl.kernel,
        out_type=jax.ShapeDtypeStruct((NC, NPAD, DP), jnp.float32),
        mesh=_sc_mesh(),
        scratch_types=[
            pltpu.VMEM((TPT, EC), jnp.int32),
            pltpu.VMEM((TPT, EC), jnp.int32),
            [pltpu.VMEM((EC, DP), jnp.float32)] * nbuf,
            [pltpu.SemaphoreType.DMA] * nbuf,
            [pltpu.SemaphoreType.DMA] * nbuf,
            pltpu.VMEM_SHARED((NPAD, DP), jnp.float32),
            pltpu.VMEM_SHARED((N, DP), jnp.float32),
        ],
        compiler_params=pltpu.CompilerParams(use_tc_tiling_on_sc=False),
    )
    def kfn(h_hbm, s_hbm, d_hbm, z_hbm, out_hbm, sidx, didx, rows, gsem,
            ssem, acc, hsp):
        c = lax.axis_index("c")
        s = lax.axis_index("s")
        pltpu.sync_copy(z_hbm, acc.at[pl.ds(s * RPT, RPT)])

        @pl.when(s == 0)
        def _():
            pltpu.sync_copy(h_hbm, hsp)

        k = c * NS + s
        start = 39 * k + jnp.minimum(k, XTR)
        ntr = 39 + (k < XTR).astype(jnp.int32)
        pltpu.sync_copy(s_hbm.at[pl.ds(start, 39)], sidx.at[pl.ds(0, 39)])
        pltpu.sync_copy(d_hbm.at[pl.ds(start, 39)], didx.at[pl.ds(0, 39)])

        @pl.when(ntr > 39)
        def _():
            pltpu.sync_copy(s_hbm.at[pl.ds(start + 39, 1)],
                            sidx.at[pl.ds(39, 1)])
            pltpu.sync_copy(d_hbm.at[pl.ds(start + 39, 1)],
                            didx.at[pl.ds(39, 1)])

        plsc.subcore_barrier()

        def gstart(j, b):
            pltpu.async_copy(hsp.at[sidx.at[j]], rows[b], gsem[b])

        def gwait(j, b):
            pltpu.make_async_copy(hsp.at[sidx.at[j]], rows[b],
                                  gsem[b]).wait()

        def sstart(j, b):
            pltpu.async_copy(rows[b], acc.at[didx.at[j]], ssem[b], add=True)

        def swait(j, b):
            pltpu.make_async_copy(rows[b], acc.at[didx.at[j]],
                                  ssem[b]).wait()

        for b in range(nbuf):
            gstart(b, b)

        def body(i, carry):
            j0 = i * nbuf
            for b in range(nbuf):
                @pl.when(j0 + b < ntr)
                def _(jb=j0 + b, bb=b):
                    gwait(jb, bb)
                    sstart(jb, bb)
            for b in range(nbuf):
                @pl.when(j0 + b < ntr)
                def _(jb=j0 + b, bb=b):
                    swait(jb, bb)

                @pl.when(jnp.logical_and(j0 + b + nbuf < ntr,
                                         i < nrounds - 1))
                def _(jb=j0 + b + nbuf, bb=b):
                    gstart(jb, bb)
            return carry

        lax.fori_loop(0, nrounds, body, 0)
        plsc.subcore_barrier()
        pltpu.sync_copy(acc.at[pl.ds(s * RPT, RPT)],
                        out_hbm.at[c, pl.ds(s * RPT, RPT)])

    return kfn(h_pad, src2d, dst2d, zrows)


def _sc_degree(dst2d, orows, z16):
    @functools.partial(
        pl.kernel,
        out_type=jax.ShapeDtypeStruct((NC, NPAD, 16), jnp.float32),
        mesh=_sc_mesh(),
        scratch_types=[
            pltpu.VMEM((TPT, EC), jnp.int32),
            pltpu.VMEM((EC, 16), jnp.float32),
            pltpu.VMEM_SHARED((NPAD, 16), jnp.float32),
        ],
        compiler_params=pltpu.CompilerParams(use_tc_tiling_on_sc=False),
    )
    def kfn(d_hbm, o_hbm, z_hbm, out_hbm, didx, ones, acc):
        c = lax.axis_index("c")
        s = lax.axis_index("s")
        pltpu.sync_copy(z_hbm, acc.at[pl.ds(s * RPT, RPT)])
        pltpu.sync_copy(o_hbm, ones)
        k = c * NS + s
        start = 39 * k + jnp.minimum(k, XTR)
        ntr = 39 + (k < XTR).astype(jnp.int32)
        pltpu.sync_copy(d_hbm.at[pl.ds(start, 39)], didx.at[pl.ds(0, 39)])

        @pl.when(ntr > 39)
        def _():
            pltpu.sync_copy(d_hbm.at[pl.ds(start + 39, 1)],
                            didx.at[pl.ds(39, 1)])

        plsc.subcore_barrier()

        def body(j, carry):
            pltpu.sync_copy(ones, acc.at[didx.at[j]], add=True)
            return carry

        lax.fori_loop(0, ntr, body, 0)
        plsc.subcore_barrier()
        pltpu.sync_copy(acc.at[pl.ds(s * RPT, RPT)],
                        out_hbm.at[c, pl.ds(s * RPT, RPT)])

    return kfn(dst2d, orows, z16)


def _aug(x40):
    n = x40.shape[0]
    return jnp.concatenate(
        [x40, jnp.ones((n, 1), jnp.float32),
         jnp.zeros((n, DP - C - 1), jnp.float32)], axis=1)


def _tc_premlp(x, W1, b1, W2, b2):
    def body(x_ref, w1_ref, b1_ref, w2_ref, b2_ref, h_ref, hs_ref):
        a = jnp.maximum(
            jnp.dot(x_ref[...], w1_ref[...],
                    preferred_element_type=jnp.float32) + b1_ref[...], 0.0)
        h = jnp.dot(a, w2_ref[...],
                    preferred_element_type=jnp.float32) + b2_ref[...]
        h_ref[...] = h
        hs_ref[...] = _aug(h)

    return pl.pallas_call(
        body,
        grid=(N // BR,),
        in_specs=[
            pl.BlockSpec((BR, F), lambda i: (i, 0)),
            pl.BlockSpec((F, HID), lambda i: (0, 0)),
            pl.BlockSpec((1, HID), lambda i: (0, 0)),
            pl.BlockSpec((HID, C), lambda i: (0, 0)),
            pl.BlockSpec((1, C), lambda i: (0, 0)),
        ],
        out_specs=[
            pl.BlockSpec((BR, C), lambda i: (i, 0)),
            pl.BlockSpec((BR, DP), lambda i: (i, 0)),
        ],
        out_shape=[
            jax.ShapeDtypeStruct((N, C), jnp.float32),
            jax.ShapeDtypeStruct((N, DP), jnp.float32),
        ],
    )(x, W1, b1.reshape(1, HID), W2, b2.reshape(1, C))


def _tc_prep(degp, h):
    def body(dp_ref, h_ref, hs_ref):
        deg = dp_ref[0, :, 0:1] + dp_ref[1, :, 0:1]
        dis, _, _ = _deg_terms(deg)
        hs_ref[...] = _aug(h_ref[...] * dis)

    return pl.pallas_call(
        body,
        grid=(N // BR,),
        in_specs=[
            pl.BlockSpec((NC, BR, 16), lambda i: (0, i, 0)),
            pl.BlockSpec((BR, C), lambda i: (i, 0)),
        ],
        out_specs=pl.BlockSpec((BR, DP), lambda i: (i, 0)),
        out_shape=jax.ShapeDtypeStruct((N, DP), jnp.float32),
    )(degp, h)


def _tc_segmax(hs, src2, dst2):
    def body(src_ref, dst_ref, hs_ref, o_ref):
        @pl.when(pl.program_id(0) == 0)
        def _():
            o_ref[...] = jnp.full((N, DP), -jnp.inf, jnp.float32)

        def step(e, carry):
            sv = src_ref[0, 0, e]
            dv = dst_ref[0, 0, e]
            row = hs_ref[pl.ds(sv, 1), :]
            o_ref[pl.ds(dv, 1), :] = jnp.maximum(o_ref[pl.ds(dv, 1), :], row)
            return carry

        lax.fori_loop(0, SEG, step, 0)

    return pl.pallas_call(
        body,
        grid=(E // SEG,),
        in_specs=[
            pl.BlockSpec((1, 1, SEG), lambda i: (i, 0, 0),
                         memory_space=pltpu.SMEM),
            pl.BlockSpec((1, 1, SEG), lambda i: (i, 0, 0),
                         memory_space=pltpu.SMEM),
            pl.BlockSpec((N, DP), lambda i: (0, 0)),
        ],
        out_specs=pl.BlockSpec((N, DP), lambda i: (0, 0)),
        out_shape=jax.ShapeDtypeStruct((N, DP), jnp.float32),
    )(src2, dst2, hs)


def _dp_specs(degp, is_max):
    if is_max:
        return degp, pl.BlockSpec((NC, BR, 16), lambda i: (0, i, 0))
    return (jnp.zeros((NC, 8, 16), jnp.float32),
            pl.BlockSpec((NC, 8, 16), lambda i: (0, 0, 0)))


def _tc_mid(p_in, degp, wvall, isym, imean, is_max):
    def body(wv_ref, p_ref, dp_ref, o_ref):
        sym = wv_ref[isym] > 0.5
        mean = wv_ref[imean] > 0.5
        if is_max:
            p48 = p_ref[...]
            p48 = jnp.where(jnp.isfinite(p48), p48, 0.0)
            deg = dp_ref[0, :, 0:1] + dp_ref[1, :, 0:1]
        else:
            p48 = p_ref[0] + p_ref[1]
            deg = p48[:, DEGC:DEGC + 1]
        dis, _, _ = _deg_terms(deg)
        pre = jnp.where(sym, dis, jnp.ones_like(deg))
        o_ref[...] = _aug(p48[:, :C] * (_post_scale(deg, sym, mean) * pre))

    p_spec = (pl.BlockSpec((BR, DP), lambda i: (i, 0)) if is_max
              else pl.BlockSpec((NC, BR, DP), lambda i: (0, i, 0)))
    dp_arr, dp_spec = _dp_specs(degp, is_max)
    return pl.pallas_call(
        body,
        grid=(N // BR,),
        in_specs=[
            pl.BlockSpec(memory_space=pltpu.SMEM),
            p_spec,
            dp_spec,
        ],
        out_specs=pl.BlockSpec((BR, DP), lambda i: (i, 0)),
        out_shape=jax.ShapeDtypeStruct((N, DP), jnp.float32),
    )(wvall, p_in, dp_arr)


def _tc_combine0(p_in, degp, xprev, Wl, bl, wvall, is_max):
    def body(wv_ref, p_ref, dp_ref, xp_ref, w_ref, b_ref, xn_ref, hs_ref):
        sym0 = wv_ref[3] > 0.5
        mean0 = wv_ref[4] > 0.5
        sym1 = wv_ref[5] > 0.5
        if is_max:
            p48 = p_ref[...]
            p48 = jnp.where(jnp.isfinite(p48), p48, 0.0)
            deg = dp_ref[0, :, 0:1] + dp_ref[1, :, 0:1]
        else:
            p48 = p_ref[0] + p_ref[1]
            deg = p48[:, DEGC:DEGC + 1]
        p = p48[:, :C] * _post_scale(deg, sym0, mean0)
        m = jnp.maximum(wv_ref[0] * p, 0.0)
        xp = xp_ref[...]
        cadd = m + xp
        ccat = (jnp.dot(m, w_ref[0:C, :], preferred_element_type=jnp.float32)
                + jnp.dot(xp, w_ref[C:, :],
                          preferred_element_type=jnp.float32)
                + b_ref[...])
        xn = wv_ref[1] * cadd + wv_ref[2] * ccat
        xn_ref[...] = xn
        dis, _, _ = _deg_terms(deg)
        pre1 = jnp.where(sym1, dis, jnp.ones_like(deg))
        hs_ref[...] = _aug(xn * pre1)

    p_spec = (pl.BlockSpec((BR, DP), lambda i: (i, 0)) if is_max
              else pl.BlockSpec((NC, BR, DP), lambda i: (0, i, 0)))
    dp_arr, dp_spec = _dp_specs(degp, is_max)
    return pl.pallas_call(
        body,
        grid=(N // BR,),
        in_specs=[
            pl.BlockSpec(memory_space=pltpu.SMEM),
            p_spec,
            dp_spec,
            pl.BlockSpec((BR, C), lambda i: (i, 0)),
            pl.BlockSpec((2 * C, C), lambda i: (0, 0)),
            pl.BlockSpec((1, C), lambda i: (0, 0)),
        ],
        out_specs=[
            pl.BlockSpec((BR, C), lambda i: (i, 0)),
            pl.BlockSpec((BR, DP), lambda i: (i, 0)),
        ],
        out_shape=[
            jax.ShapeDtypeStruct((N, C), jnp.float32),
            jax.ShapeDtypeStruct((N, DP), jnp.float32),
        ],
    )(wvall, p_in, dp_arr, xprev, Wl, bl.reshape(1, C))


def _tc_comb1jk(p_in, degp, x1, h0, Wl, bl, jk_W, jk_b, wvall, is_max):
    def body(wv_ref, p_ref, dp_ref, x1_ref, h_ref, w_ref, b_ref,
             jw_w_ref, jw_b_ref, o_ref):
        sym1 = wv_ref[5] > 0.5
        mean1 = wv_ref[9] > 0.5
        if is_max:
            p48 = p_ref[...]
            p48 = jnp.where(jnp.isfinite(p48), p48, 0.0)
            deg = dp_ref[0, :, 0:1] + dp_ref[1, :, 0:1]
        else:
            p48 = p_ref[0] + p_ref[1]
            deg = p48[:, DEGC:DEGC + 1]
        p = p48[:, :C] * _post_scale(deg, sym1, mean1)
        m = jnp.maximum(wv_ref[6] * p, 0.0)
        x1 = x1_ref[...]
        cadd = m + x1
        ccat = (jnp.dot(m, w_ref[0:C, :], preferred_element_type=jnp.float32)
                + jnp.dot(x1, w_ref[C:, :],
                          preferred_element_type=jnp.float32)
                + b_ref[...])
        x2 = wv_ref[7] * cadd + wv_ref[8] * ccat
        h = h_ref[...]
        cat = (jnp.dot(h, jw_w_ref[0:C, :], preferred_element_type=jnp.float32)
               + jnp.dot(x1, jw_w_ref[C:2 * C, :],
                         preferred_element_type=jnp.float32)
               + jnp.dot(x2, jw_w_ref[2 * C:, :],
                         preferred_element_type=jnp.float32)
               + jw_b_ref[...])
        mx = jnp.maximum(jnp.maximum(h, x1), x2)
        mn = (h + x1 + x2) / 3.0
        lin = (wv_ref[10] * x2 + wv_ref[11] * mx + wv_ref[12] * mn
               + wv_ref[13] * cat)
        rmax = jnp.max(lin, axis=1, keepdims=True)
        sh = lin - rmax
        o_ref[...] = sh - jnp.log(jnp.sum(jnp.exp(sh), axis=1, keepdims=True))

    p_spec = (pl.BlockSpec((BR, DP), lambda i: (i, 0)) if is_max
              else pl.BlockSpec((NC, BR, DP), lambda i: (0, i, 0)))
    dp_arr, dp_spec = _dp_specs(degp, is_max)
    return pl.pallas_call(
        body,
        grid=(N // BR,),
        in_specs=[
            pl.BlockSpec(memory_space=pltpu.SMEM),
            p_spec,
            dp_spec,
            pl.BlockSpec((BR, C), lambda i: (i, 0)),
            pl.BlockSpec((BR, C), lambda i: (i, 0)),
            pl.BlockSpec((2 * C, C), lambda i: (0, 0)),
            pl.BlockSpec((1, C), lambda i: (0, 0)),
            pl.BlockSpec((3 * C, C), lambda i: (0, 0)),
            pl.BlockSpec((1, C), lambda i: (0, 0)),
        ],
        out_specs=pl.BlockSpec((BR, C), lambda i: (i, 0)),
        out_shape=jax.ShapeDtypeStruct((N, C), jnp.float32),
    )(wvall, p_in, dp_arr, x1, h0, Wl, bl.reshape(1, C), jk_W,
      jk_b.reshape(1, C))


def _gumbel_u9():
    import numpy as _np
    hexes = [
        ['0x1.0f7e560000000p-1', '0x1.40e2180000000p-2',
         '0x1.0000000000000p-1', '0x1.0000000000000p-1'],
        ['0x1.cd95440000000p-1', '0x1.658bd60000000p-1',
         '0x1.0000000000000p-1', '0x1.0000000000000p-1'],
        ['0x1.7490580000000p-1', '0x1.93634c0000000p-1',
         '0x1.741c740000000p-3', '0x1.0000000000000p-1'],
        ['0x1.0cef100000000p-2', '0x1.c58cf00000000p-4',
         '0x1.9efd300000000p-3', '0x1.0000000000000p-1'],
        ['0x1.55a0840000000p-1', '0x1.7166a40000000p-1',
         '0x1.0000000000000p-1', '0x1.0000000000000p-1'],
        ['0x1.03ad540000000p-3', '0x1.6ac7a20000000p-2',
         '0x1.0000000000000p-1', '0x1.0000000000000p-1'],
        ['0x1.8e593e0000000p-2', '0x1.5f2d9e0000000p-4',
         '0x1.0000000000000p-1', '0x1.0000000000000p-1'],
        ['0x1.a1f7a20000000p-5', '0x1.3e51ec0000000p-1',
         '0x1.0000000000000p-1', '0x1.0000000000000p-1'],
        ['0x1.7ccb5e0000000p-1', '0x1.23408a0000000p-1',
         '0x1.e40bf20000000p-1', '0x1.8f7e0c0000000p-1'],
    ]
    return _np.array([[float.fromhex(v) for v in row] for row in hexes],
                     dtype=_np.float32)


def _g9_const():
    import numpy as _np
    u = _gumbel_u9()
    return _np.log(-_np.log(u)).astype(_np.float32)


_G9 = _g9_const()


def _alpha9(neigh_alphas, aggr_alphas, norm_alphas, comb_alphas, jk_alphas):
    def padneg(a):
        r, k = a.shape
        if k == 4:
            return a
        return jnp.concatenate(
            [a, jnp.full((r, 4 - k), -jnp.inf, a.dtype)], 1)

    return jnp.concatenate([
        padneg(neigh_alphas), padneg(aggr_alphas), padneg(norm_alphas),
        padneg(comb_alphas), padneg(jk_alphas)], axis=0)


def kernel(x, edge_index, pre_W1, pre_b1, pre_W2, pre_b2, comb_W, comb_b,
           jk_W, jk_b, neigh_alphas, aggr_alphas, norm_alphas, comb_alphas,
           jk_alphas):
    g9 = jnp.asarray(_G9)

    a9 = _alpha9(neigh_alphas, aggr_alphas, norm_alphas, comb_alphas,
                 jk_alphas)
    ws9 = jax.nn.softmax((a9 - g9) / TEMP, axis=-1)
    argm = jnp.argmax(ws9, axis=-1)
    oh9 = jax.nn.one_hot(argm, 4, dtype=ws9.dtype)
    masks9 = (oh9 - ws9) + ws9
    rs = jnp.sum(masks9, axis=-1)

    sym0 = (argm[4] == 0).astype(jnp.float32)
    sym1 = (argm[5] == 0).astype(jnp.float32)
    mean0 = (argm[2] == 1).astype(jnp.float32)
    mean1 = (argm[3] == 1).astype(jnp.float32)
    wprod0 = (rs[0] * rs[2]) * rs[4]
    wprod1 = (rs[1] * rs[3]) * rs[5]
    wvall = jnp.concatenate([
        jnp.stack([wprod0, masks9[6, 0], masks9[6, 1], sym0, mean0, sym1,
                   wprod1, masks9[7, 0], masks9[7, 1], mean1]),
        masks9[8], jnp.zeros((2,), jnp.float32)])

    src2d = edge_index[0].reshape(ETR, EC)
    dst2d = edge_index[1].reshape(ETR, EC)
    zrows = jnp.zeros((RPT, DP), jnp.float32)

    h0, hs0 = _tc_premlp(x, pre_W1, pre_b1, pre_W2, pre_b2)

    def cold_inputs():
        return (edge_index[0].reshape(E // SEG, 1, SEG),
                edge_index[1].reshape(E // SEG, 1, SEG),
                jnp.ones((EC, 16), jnp.float32),
                jnp.zeros((RPT, 16), jnp.float32))


    def l0_rw_1h_sum():
        parts = _sc_prop_sum(hs0, src2d, dst2d, zrows)
        return _tc_combine0(parts, None, h0, comb_W[0], comb_b[0], wvall,
                            False)

    def l0_rw_1h_max():
        srcseg, dstseg, orows, z16 = cold_inputs()
        degp = _sc_degree(dst2d, orows, z16)
        pm = _tc_segmax(hs0, srcseg, dstseg)
        return _tc_combine0(pm, degp, h0, comb_W[0], comb_b[0], wvall, True)

    def l0_rw_2h_sum():
        p1 = _sc_prop_sum(hs0, src2d, dst2d, zrows)
        hmid = _tc_mid(p1, None, wvall, 3, 4, False)
        p2 = _sc_prop_sum(hmid, src2d, dst2d, zrows)
        return _tc_combine0(p2, None, h0, comb_W[0], comb_b[0], wvall,
                            False)

    def l0_rw_2h_max():
        srcseg, dstseg, orows, z16 = cold_inputs()
        degp = _sc_degree(dst2d, orows, z16)
        pm1 = _tc_segmax(hs0, srcseg, dstseg)
        hmid = _tc_mid(pm1, degp, wvall, 3, 4, True)
        pm2 = _tc_segmax(hmid, srcseg, dstseg)
        return _tc_combine0(pm2, degp, h0, comb_W[0], comb_b[0], wvall,
                            True)

    def l0_sym_1h_sum():
        _, _, orows, z16 = cold_inputs()
        degp = _sc_degree(dst2d, orows, z16)
        hss = _tc_prep(degp, h0)
        parts = _sc_prop_sum(hss, src2d, dst2d, zrows)
        return _tc_combine0(parts, None, h0, comb_W[0], comb_b[0], wvall,
                            False)

    def l0_sym_1h_max():
        srcseg, dstseg, orows, z16 = cold_inputs()
        degp = _sc_degree(dst2d, orows, z16)
        hss = _tc_prep(degp, h0)
        pm = _tc_segmax(hss, srcseg, dstseg)
        return _tc_combine0(pm, degp, h0, comb_W[0], comb_b[0], wvall, True)

    def l0_sym_2h_sum():
        _, _, orows, z16 = cold_inputs()
        degp = _sc_degree(dst2d, orows, z16)
        hss = _tc_prep(degp, h0)
        p1 = _sc_prop_sum(hss, src2d, dst2d, zrows)
        hmid = _tc_mid(p1, None, wvall, 3, 4, False)
        p2 = _sc_prop_sum(hmid, src2d, dst2d, zrows)
        return _tc_combine0(p2, None, h0, comb_W[0], comb_b[0], wvall,
                            False)

    def l0_sym_2h_max():
        srcseg, dstseg, orows, z16 = cold_inputs()
        degp = _sc_degree(dst2d, orows, z16)
        hss = _tc_prep(degp, h0)
        pm1 = _tc_segmax(hss, srcseg, dstseg)
        hmid = _tc_mid(pm1, degp, wvall, 3, 4, True)
        pm2 = _tc_segmax(hmid, srcseg, dstseg)
        return _tc_combine0(pm2, degp, h0, comb_W[0], comb_b[0], wvall,
                            True)

    bi0 = (4 * (argm[4] == 0).astype(jnp.int32)
           + 2 * (argm[0] == 1).astype(jnp.int32)
           + (argm[2] == 2).astype(jnp.int32))
    x1, hs1 = lax.switch(bi0, [
        l0_rw_1h_sum, l0_rw_1h_max, l0_rw_2h_sum, l0_rw_2h_max,
        l0_sym_1h_sum, l0_sym_1h_max, l0_sym_2h_sum, l0_sym_2h_max,
    ])


    def l1_1h_sum():
        parts = _sc_prop_sum(hs1, src2d, dst2d, zrows)
        return _tc_comb1jk(parts, None, x1, h0, comb_W[1], comb_b[1],
                           jk_W, jk_b, wvall, False)

    def l1_1h_max():
        srcseg, dstseg, orows, z16 = cold_inputs()
        degp = _sc_degree(dst2d, orows, z16)
        pm = _tc_segmax(hs1, srcseg, dstseg)
        return _tc_comb1jk(pm, degp, x1, h0, comb_W[1], comb_b[1],
                           jk_W, jk_b, wvall, True)

    def l1_2h_sum():
        p1 = _sc_prop_sum(hs1, src2d, dst2d, zrows)
        hmid = _tc_mid(p1, None, wvall, 5, 9, False)
        p2 = _sc_prop_sum(hmid, src2d, dst2d, zrows)
        return _tc_comb1jk(p2, None, x1, h0, comb_W[1], comb_b[1],
                           jk_W, jk_b, wvall, False)

    def l1_2h_max():
        srcseg, dstseg, orows, z16 = cold_inputs()
        degp = _sc_degree(dst2d, orows, z16)
        pm1 = _tc_segmax(hs1, srcseg, dstseg)
        hmid = _tc_mid(pm1, degp, wvall, 5, 9, True)
        pm2 = _tc_segmax(hmid, srcseg, dstseg)
        return _tc_comb1jk(pm2, degp, x1, h0, comb_W[1], comb_b[1],
                           jk_W, jk_b, wvall, True)

    bi1 = (2 * (argm[1] == 1).astype(jnp.int32)
           + (argm[3] == 2).astype(jnp.int32))
    return lax.switch(bi1, [l1_1h_sum, l1_1h_max, l1_2h_sum, l1_2h_max])


# speedup vs baseline: 1.1503x; 1.0245x over previous
import functools

import jax
import jax.numpy as jnp
from jax import lax
from jax.experimental import pallas as pl
from jax.experimental.pallas import tpu as pltpu
from jax.experimental.pallas import tpu_sc as plsc

N = 10000
E = 160000
F = 128
HID = 256
C = 40
DP = 48
DEGC = C
NLAYERS = 2
TEMP = 0.5

NC, NS = 2, 16
NW = NC * NS
EC = 128
ETR = E // EC
TPT = 40
NPAD = 10112
RPT = NPAD // NS
BR = 2000
SEG = 1600
XTR = ETR - 39 * NW


def _deg_terms(deg):
    pos = deg > 0.0
    dmax = jnp.maximum(deg, 1e-12)
    dis = jnp.where(pos, lax.rsqrt(dmax), 0.0)
    dinv = jnp.where(pos, 1.0 / dmax, 0.0)
    inv1 = 1.0 / jnp.maximum(deg, 1.0)
    return dis, dinv, inv1


def _post_scale(deg, sym, mean):
    dis, dinv, inv1 = _deg_terms(deg)
    one = jnp.ones_like(deg)
    return jnp.where(sym, dis, dinv) * jnp.where(mean, inv1, one)


def _sc_mesh():
    return plsc.VectorSubcoreMesh(core_axis_name="c", subcore_axis_name="s",
                                  num_cores=NC, num_subcores=NS)


def _sc_prop_sum(h_pad, ei3, zrows):
    nbuf = 8
    nrounds = TPT // nbuf

    @functools.partial(
        pl.kernel,
        out_type=jax.ShapeDtypeStruct((NC, NPAD, DP), jnp.float32),
        mesh=_sc_mesh(),
        scratch_types=[
            pltpu.VMEM((TPT, EC), jnp.int32),
            pltpu.VMEM((TPT, EC), jnp.int32),
            [pltpu.VMEM((EC, DP), jnp.float32)] * nbuf,
            [pltpu.SemaphoreType.DMA] * nbuf,
            [pltpu.SemaphoreType.DMA] * nbuf,
            pltpu.VMEM_SHARED((NPAD, DP), jnp.float32),
            pltpu.VMEM_SHARED((N, DP), jnp.float32),
        ],
        compiler_params=pltpu.CompilerParams(use_tc_tiling_on_sc=False),
    )
    def kfn(h_hbm, ei_hbm, z_hbm, out_hbm, sidx, didx, rows, gsem,
            ssem, acc, hsp):
        c = lax.axis_index("c")
        s = lax.axis_index("s")
        pltpu.sync_copy(z_hbm, acc.at[pl.ds(s * RPT, RPT)])

        @pl.when(s == 0)
        def _():
            pltpu.sync_copy(h_hbm, hsp)

        k = c * NS + s
        start = 39 * k + jnp.minimum(k, XTR)
        ntr = 39 + (k < XTR).astype(jnp.int32)
        pltpu.sync_copy(ei_hbm.at[0, pl.ds(start, 39)],
                        sidx.at[pl.ds(0, 39)])
        pltpu.sync_copy(ei_hbm.at[1, pl.ds(start, 39)],
                        didx.at[pl.ds(0, 39)])

        @pl.when(ntr > 39)
        def _():
            pltpu.sync_copy(ei_hbm.at[0, pl.ds(start + 39, 1)],
                            sidx.at[pl.ds(39, 1)])
            pltpu.sync_copy(ei_hbm.at[1, pl.ds(start + 39, 1)],
                            didx.at[pl.ds(39, 1)])

        plsc.subcore_barrier()

        def gstart(j, b):
            pltpu.async_copy(hsp.at[sidx.at[j]], rows[b], gsem[b])

        def gwait(j, b):
            pltpu.make_async_copy(hsp.at[sidx.at[j]], rows[b],
                                  gsem[b]).wait()

        def sstart(j, b):
            pltpu.async_copy(rows[b], acc.at[didx.at[j]], ssem[b], add=True)

        def swait(j, b):
            pltpu.make_async_copy(rows[b], acc.at[didx.at[j]],
                                  ssem[b]).wait()

        for b in range(nbuf):
            gstart(b, b)

        def body(i, carry):
            j0 = i * nbuf
            for b in range(nbuf):
                @pl.when(j0 + b < ntr)
                def _(jb=j0 + b, bb=b):
                    gwait(jb, bb)
                    sstart(jb, bb)
            for b in range(nbuf):
                @pl.when(j0 + b < ntr)
                def _(jb=j0 + b, bb=b):
                    swait(jb, bb)

                @pl.when(jnp.logical_and(j0 + b + nbuf < ntr,
                                         i < nrounds - 1))
                def _(jb=j0 + b + nbuf, bb=b):
                    gstart(jb, bb)
            return carry

        lax.fori_loop(0, nrounds, body, 0)
        plsc.subcore_barrier()
        pltpu.sync_copy(acc.at[pl.ds(s * RPT, RPT)],
                        out_hbm.at[c, pl.ds(s * RPT, RPT)])

    return kfn(h_pad, ei3, zrows)


def _sc_degree(ei3, orows, z16):
    @functools.partial(
        pl.kernel,
        out_type=jax.ShapeDtypeStruct((NC, NPAD, 16), jnp.float32),
        mesh=_sc_mesh(),
        scratch_types=[
            pltpu.VMEM((TPT, EC), jnp.int32),
            pltpu.VMEM((EC, 16), jnp.float32),
            pltpu.VMEM_SHARED((NPAD, 16), jnp.float32),
        ],
        compiler_params=pltpu.CompilerParams(use_tc_tiling_on_sc=False),
    )
    def kfn(ei_hbm, o_hbm, z_hbm, out_hbm, didx, ones, acc):
        c = lax.axis_index("c")
        s = lax.axis_index("s")
        pltpu.sync_copy(z_hbm, acc.at[pl.ds(s * RPT, RPT)])
        pltpu.sync_copy(o_hbm, ones)
        k = c * NS + s
        start = 39 * k + jnp.minimum(k, XTR)
        ntr = 39 + (k < XTR).astype(jnp.int32)
        pltpu.sync_copy(ei_hbm.at[1, pl.ds(start, 39)],
                        didx.at[pl.ds(0, 39)])

        @pl.when(ntr > 39)
        def _():
            pltpu.sync_copy(ei_hbm.at[1, pl.ds(start + 39, 1)],
                            didx.at[pl.ds(39, 1)])

        plsc.subcore_barrier()

        def body(j, carry):
            pltpu.sync_copy(ones, acc.at[didx.at[j]], add=True)
            return carry

        lax.fori_loop(0, ntr, body, 0)
        plsc.subcore_barrier()
        pltpu.sync_copy(acc.at[pl.ds(s * RPT, RPT)],
                        out_hbm.at[c, pl.ds(s * RPT, RPT)])

    return kfn(ei3, orows, z16)


def _aug(x40):
    n = x40.shape[0]
    return jnp.concatenate(
        [x40, jnp.ones((n, 1), jnp.float32),
         jnp.zeros((n, DP - C - 1), jnp.float32)], axis=1)


def _tc_premlp(x, W1, b1, W2, b2):
    def body(x_ref, w1_ref, b1_ref, w2_ref, b2_ref, h_ref, hs_ref):
        a = jnp.maximum(
            jnp.dot(x_ref[...], w1_ref[...],
                    preferred_element_type=jnp.float32) + b1_ref[...], 0.0)
        h = jnp.dot(a, w2_ref[...],
                    preferred_element_type=jnp.float32) + b2_ref[...]
        h_ref[...] = h
        hs_ref[...] = _aug(h)

    return pl.pallas_call(
        body,
        grid=(N // BR,),
        in_specs=[
            pl.BlockSpec((BR, F), lambda i: (i, 0)),
            pl.BlockSpec((F, HID), lambda i: (0, 0)),
            pl.BlockSpec((1, HID), lambda i: (0, 0)),
            pl.BlockSpec((HID, C), lambda i: (0, 0)),
            pl.BlockSpec((1, C), lambda i: (0, 0)),
        ],
        out_specs=[
            pl.BlockSpec((BR, C), lambda i: (i, 0)),
            pl.BlockSpec((BR, DP), lambda i: (i, 0)),
        ],
        out_shape=[
            jax.ShapeDtypeStruct((N, C), jnp.float32),
            jax.ShapeDtypeStruct((N, DP), jnp.float32),
        ],
    )(x, W1, b1.reshape(1, HID), W2, b2.reshape(1, C))


def _tc_prep(degp, h):
    def body(dp_ref, h_ref, hs_ref):
        deg = dp_ref[0, :, 0:1] + dp_ref[1, :, 0:1]
        dis, _, _ = _deg_terms(deg)
        hs_ref[...] = _aug(h_ref[...] * dis)

    return pl.pallas_call(
        body,
        grid=(N // BR,),
        in_specs=[
            pl.BlockSpec((NC, BR, 16), lambda i: (0, i, 0)),
            pl.BlockSpec((BR, C), lambda i: (i, 0)),
        ],
        out_specs=pl.BlockSpec((BR, DP), lambda i: (i, 0)),
        out_shape=jax.ShapeDtypeStruct((N, DP), jnp.float32),
    )(degp, h)


def _tc_segmax(hs, src2, dst2):
    def body(src_ref, dst_ref, hs_ref, o_ref):
        @pl.when(pl.program_id(0) == 0)
        def _():
            o_ref[...] = jnp.full((N, DP), -jnp.inf, jnp.float32)

        def step(e, carry):
            sv = src_ref[0, 0, e]
            dv = dst_ref[0, 0, e]
            row = hs_ref[pl.ds(sv, 1), :]
            o_ref[pl.ds(dv, 1), :] = jnp.maximum(o_ref[pl.ds(dv, 1), :], row)
            return carry

        lax.fori_loop(0, SEG, step, 0)

    return pl.pallas_call(
        body,
        grid=(E // SEG,),
        in_specs=[
            pl.BlockSpec((1, 1, SEG), lambda i: (i, 0, 0),
                         memory_space=pltpu.SMEM),
            pl.BlockSpec((1, 1, SEG), lambda i: (i, 0, 0),
                         memory_space=pltpu.SMEM),
            pl.BlockSpec((N, DP), lambda i: (0, 0)),
        ],
        out_specs=pl.BlockSpec((N, DP), lambda i: (0, 0)),
        out_shape=jax.ShapeDtypeStruct((N, DP), jnp.float32),
    )(src2, dst2, hs)


def _dp_specs(degp, is_max):
    if is_max:
        return degp, pl.BlockSpec((NC, BR, 16), lambda i: (0, i, 0))
    return (jnp.zeros((NC, 8, 16), jnp.float32),
            pl.BlockSpec((NC, 8, 16), lambda i: (0, 0, 0)))


def _tc_mid(p_in, degp, wvall, isym, imean, is_max):
    def body(wv_ref, p_ref, dp_ref, o_ref):
        sym = wv_ref[isym] > 0.5
        mean = wv_ref[imean] > 0.5
        if is_max:
            p48 = p_ref[...]
            p48 = jnp.where(jnp.isfinite(p48), p48, 0.0)
            deg = dp_ref[0, :, 0:1] + dp_ref[1, :, 0:1]
        else:
            p48 = p_ref[0] + p_ref[1]
            deg = p48[:, DEGC:DEGC + 1]
        dis, _, _ = _deg_terms(deg)
        pre = jnp.where(sym, dis, jnp.ones_like(deg))
        o_ref[...] = _aug(p48[:, :C] * (_post_scale(deg, sym, mean) * pre))

    p_spec = (pl.BlockSpec((BR, DP), lambda i: (i, 0)) if is_max
              else pl.BlockSpec((NC, BR, DP), lambda i: (0, i, 0)))
    dp_arr, dp_spec = _dp_specs(degp, is_max)
    return pl.pallas_call(
        body,
        grid=(N // BR,),
        in_specs=[
            pl.BlockSpec(memory_space=pltpu.SMEM),
            p_spec,
            dp_spec,
        ],
        out_specs=pl.BlockSpec((BR, DP), lambda i: (i, 0)),
        out_shape=jax.ShapeDtypeStruct((N, DP), jnp.float32),
    )(wvall, p_in, dp_arr)


def _tc_combine0(p_in, degp, xprev, Wl, bl, wvall, is_max):
    def body(wv_ref, p_ref, dp_ref, xp_ref, w_ref, b_ref, xn_ref, hs_ref):
        sym0 = wv_ref[3] > 0.5
        mean0 = wv_ref[4] > 0.5
        sym1 = wv_ref[5] > 0.5
        if is_max:
            p48 = p_ref[...]
            p48 = jnp.where(jnp.isfinite(p48), p48, 0.0)
            deg = dp_ref[0, :, 0:1] + dp_ref[1, :, 0:1]
        else:
            p48 = p_ref[0] + p_ref[1]
            deg = p48[:, DEGC:DEGC + 1]
        p = p48[:, :C] * _post_scale(deg, sym0, mean0)
        m = jnp.maximum(wv_ref[0] * p, 0.0)
        xp = xp_ref[...]
        cadd = m + xp
        ccat = (jnp.dot(m, w_ref[0:C, :], preferred_element_type=jnp.float32)
                + jnp.dot(xp, w_ref[C:, :],
                          preferred_element_type=jnp.float32)
                + b_ref[...])
        xn = wv_ref[1] * cadd + wv_ref[2] * ccat
        xn_ref[...] = xn
        dis, _, _ = _deg_terms(deg)
        pre1 = jnp.where(sym1, dis, jnp.ones_like(deg))
        hs_ref[...] = _aug(xn * pre1)

    p_spec = (pl.BlockSpec((BR, DP), lambda i: (i, 0)) if is_max
              else pl.BlockSpec((NC, BR, DP), lambda i: (0, i, 0)))
    dp_arr, dp_spec = _dp_specs(degp, is_max)
    return pl.pallas_call(
        body,
        grid=(N // BR,),
        in_specs=[
            pl.BlockSpec(memory_space=pltpu.SMEM),
            p_spec,
            dp_spec,
            pl.BlockSpec((BR, C), lambda i: (i, 0)),
            pl.BlockSpec((2 * C, C), lambda i: (0, 0)),
            pl.BlockSpec((1, C), lambda i: (0, 0)),
        ],
        out_specs=[
            pl.BlockSpec((BR, C), lambda i: (i, 0)),
            pl.BlockSpec((BR, DP), lambda i: (i, 0)),
        ],
        out_shape=[
            jax.ShapeDtypeStruct((N, C), jnp.float32),
            jax.ShapeDtypeStruct((N, DP), jnp.float32),
        ],
    )(wvall, p_in, dp_arr, xprev, Wl, bl.reshape(1, C))


def _tc_comb1jk(p_in, degp, x1, h0, Wl, bl, jk_W, jk_b, wvall, is_max):
    def body(wv_ref, p_ref, dp_ref, x1_ref, h_ref, w_ref, b_ref,
             jw_w_ref, jw_b_ref, o_ref):
        sym1 = wv_ref[5] > 0.5
        mean1 = wv_ref[9] > 0.5
        if is_max:
            p48 = p_ref[...]
            p48 = jnp.where(jnp.isfinite(p48), p48, 0.0)
            deg = dp_ref[0, :, 0:1] + dp_ref[1, :, 0:1]
        else:
            p48 = p_ref[0] + p_ref[1]
            deg = p48[:, DEGC:DEGC + 1]
        p = p48[:, :C] * _post_scale(deg, sym1, mean1)
        m = jnp.maximum(wv_ref[6] * p, 0.0)
        x1 = x1_ref[...]
        cadd = m + x1
        ccat = (jnp.dot(m, w_ref[0:C, :], preferred_element_type=jnp.float32)
                + jnp.dot(x1, w_ref[C:, :],
                          preferred_element_type=jnp.float32)
                + b_ref[...])
        x2 = wv_ref[7] * cadd + wv_ref[8] * ccat
        h = h_ref[...]
        cat = (jnp.dot(h, jw_w_ref[0:C, :], preferred_element_type=jnp.float32)
               + jnp.dot(x1, jw_w_ref[C:2 * C, :],
                         preferred_element_type=jnp.float32)
               + jnp.dot(x2, jw_w_ref[2 * C:, :],
                         preferred_element_type=jnp.float32)
               + jw_b_ref[...])
        mx = jnp.maximum(jnp.maximum(h, x1), x2)
        mn = (h + x1 + x2) / 3.0
        lin = (wv_ref[10] * x2 + wv_ref[11] * mx + wv_ref[12] * mn
               + wv_ref[13] * cat)
        rmax = jnp.max(lin, axis=1, keepdims=True)
        sh = lin - rmax
        o_ref[...] = sh - jnp.log(jnp.sum(jnp.exp(sh), axis=1, keepdims=True))

    p_spec = (pl.BlockSpec((BR, DP), lambda i: (i, 0)) if is_max
              else pl.BlockSpec((NC, BR, DP), lambda i: (0, i, 0)))
    dp_arr, dp_spec = _dp_specs(degp, is_max)
    return pl.pallas_call(
        body,
        grid=(N // BR,),
        in_specs=[
            pl.BlockSpec(memory_space=pltpu.SMEM),
            p_spec,
            dp_spec,
            pl.BlockSpec((BR, C), lambda i: (i, 0)),
            pl.BlockSpec((BR, C), lambda i: (i, 0)),
            pl.BlockSpec((2 * C, C), lambda i: (0, 0)),
            pl.BlockSpec((1, C), lambda i: (0, 0)),
            pl.BlockSpec((3 * C, C), lambda i: (0, 0)),
            pl.BlockSpec((1, C), lambda i: (0, 0)),
        ],
        out_specs=pl.BlockSpec((BR, C), lambda i: (i, 0)),
        out_shape=jax.ShapeDtypeStruct((N, C), jnp.float32),
    )(wvall, p_in, dp_arr, x1, h0, Wl, bl.reshape(1, C), jk_W,
      jk_b.reshape(1, C))


def _gumbel_u9():
    import numpy as _np
    hexes = [
        ['0x1.0f7e560000000p-1', '0x1.40e2180000000p-2',
         '0x1.0000000000000p-1', '0x1.0000000000000p-1'],
        ['0x1.cd95440000000p-1', '0x1.658bd60000000p-1',
         '0x1.0000000000000p-1', '0x1.0000000000000p-1'],
        ['0x1.7490580000000p-1', '0x1.93634c0000000p-1',
         '0x1.741c740000000p-3', '0x1.0000000000000p-1'],
        ['0x1.0cef100000000p-2', '0x1.c58cf00000000p-4',
         '0x1.9efd300000000p-3', '0x1.0000000000000p-1'],
        ['0x1.55a0840000000p-1', '0x1.7166a40000000p-1',
         '0x1.0000000000000p-1', '0x1.0000000000000p-1'],
        ['0x1.03ad540000000p-3', '0x1.6ac7a20000000p-2',
         '0x1.0000000000000p-1', '0x1.0000000000000p-1'],
        ['0x1.8e593e0000000p-2', '0x1.5f2d9e0000000p-4',
         '0x1.0000000000000p-1', '0x1.0000000000000p-1'],
        ['0x1.a1f7a20000000p-5', '0x1.3e51ec0000000p-1',
         '0x1.0000000000000p-1', '0x1.0000000000000p-1'],
        ['0x1.7ccb5e0000000p-1', '0x1.23408a0000000p-1',
         '0x1.e40bf20000000p-1', '0x1.8f7e0c0000000p-1'],
    ]
    return _np.array([[float.fromhex(v) for v in row] for row in hexes],
                     dtype=_np.float32)


def _g9_const():
    import numpy as _np
    u = _gumbel_u9()
    return _np.log(-_np.log(u)).astype(_np.float32)


_G9 = _g9_const()


def _alpha9(neigh_alphas, aggr_alphas, norm_alphas, comb_alphas, jk_alphas):
    def padneg(a):
        r, k = a.shape
        if k == 4:
            return a
        return jnp.concatenate(
            [a, jnp.full((r, 4 - k), -jnp.inf, a.dtype)], 1)

    return jnp.concatenate([
        padneg(neigh_alphas), padneg(aggr_alphas), padneg(norm_alphas),
        padneg(comb_alphas), padneg(jk_alphas)], axis=0)


def kernel(x, edge_index, pre_W1, pre_b1, pre_W2, pre_b2, comb_W, comb_b,
           jk_W, jk_b, neigh_alphas, aggr_alphas, norm_alphas, comb_alphas,
           jk_alphas):
    g9 = jnp.asarray(_G9)

    a9 = _alpha9(neigh_alphas, aggr_alphas, norm_alphas, comb_alphas,
                 jk_alphas)
    ws9 = jax.nn.softmax((a9 - g9) / TEMP, axis=-1)
    argm = jnp.argmax(ws9, axis=-1)
    oh9 = jax.nn.one_hot(argm, 4, dtype=ws9.dtype)
    masks9 = (oh9 - ws9) + ws9
    rs = jnp.sum(masks9, axis=-1)

    sym0 = (argm[4] == 0).astype(jnp.float32)
    sym1 = (argm[5] == 0).astype(jnp.float32)
    mean0 = (argm[2] == 1).astype(jnp.float32)
    mean1 = (argm[3] == 1).astype(jnp.float32)
    wprod0 = (rs[0] * rs[2]) * rs[4]
    wprod1 = (rs[1] * rs[3]) * rs[5]
    wvall = jnp.concatenate([
        jnp.stack([wprod0, masks9[6, 0], masks9[6, 1], sym0, mean0, sym1,
                   wprod1, masks9[7, 0], masks9[7, 1], mean1]),
        masks9[8], jnp.zeros((2,), jnp.float32)])

    ei3 = edge_index.reshape(2, ETR, EC)
    zrows = jnp.zeros((RPT, DP), jnp.float32)

    h0, hs0 = _tc_premlp(x, pre_W1, pre_b1, pre_W2, pre_b2)

    def cold_inputs():
        return (edge_index[0].reshape(E // SEG, 1, SEG),
                edge_index[1].reshape(E // SEG, 1, SEG),
                jnp.ones((EC, 16), jnp.float32),
                jnp.zeros((RPT, 16), jnp.float32))


    def l0_rw_1h_sum():
        parts = _sc_prop_sum(hs0, ei3, zrows)
        return _tc_combine0(parts, None, h0, comb_W[0], comb_b[0], wvall,
                            False)

    def l0_rw_1h_max():
        srcseg, dstseg, orows, z16 = cold_inputs()
        degp = _sc_degree(ei3, orows, z16)
        pm = _tc_segmax(hs0, srcseg, dstseg)
        return _tc_combine0(pm, degp, h0, comb_W[0], comb_b[0], wvall, True)

    def l0_rw_2h_sum():
        p1 = _sc_prop_sum(hs0, ei3, zrows)
        hmid = _tc_mid(p1, None, wvall, 3, 4, False)
        p2 = _sc_prop_sum(hmid, ei3, zrows)
        return _tc_combine0(p2, None, h0, comb_W[0], comb_b[0], wvall,
                            False)

    def l0_rw_2h_max():
        srcseg, dstseg, orows, z16 = cold_inputs()
        degp = _sc_degree(ei3, orows, z16)
        pm1 = _tc_segmax(hs0, srcseg, dstseg)
        hmid = _tc_mid(pm1, degp, wvall, 3, 4, True)
        pm2 = _tc_segmax(hmid, srcseg, dstseg)
        return _tc_combine0(pm2, degp, h0, comb_W[0], comb_b[0], wvall,
                            True)

    def l0_sym_1h_sum():
        _, _, orows, z16 = cold_inputs()
        degp = _sc_degree(ei3, orows, z16)
        hss = _tc_prep(degp, h0)
        parts = _sc_prop_sum(hss, ei3, zrows)
        return _tc_combine0(parts, None, h0, comb_W[0], comb_b[0], wvall,
                            False)

    def l0_sym_1h_max():
        srcseg, dstseg, orows, z16 = cold_inputs()
        degp = _sc_degree(ei3, orows, z16)
        hss = _tc_prep(degp, h0)
        pm = _tc_segmax(hss, srcseg, dstseg)
        return _tc_combine0(pm, degp, h0, comb_W[0], comb_b[0], wvall, True)

    def l0_sym_2h_sum():
        _, _, orows, z16 = cold_inputs()
        degp = _sc_degree(ei3, orows, z16)
        hss = _tc_prep(degp, h0)
        p1 = _sc_prop_sum(hss, ei3, zrows)
        hmid = _tc_mid(p1, None, wvall, 3, 4, False)
        p2 = _sc_prop_sum(hmid, ei3, zrows)
        return _tc_combine0(p2, None, h0, comb_W[0], comb_b[0], wvall,
                            False)

    def l0_sym_2h_max():
        srcseg, dstseg, orows, z16 = cold_inputs()
        degp = _sc_degree(ei3, orows, z16)
        hss = _tc_prep(degp, h0)
        pm1 = _tc_segmax(hss, srcseg, dstseg)
        hmid = _tc_mid(pm1, degp, wvall, 3, 4, True)
        pm2 = _tc_segmax(hmid, srcseg, dstseg)
        return _tc_combine0(pm2, degp, h0, comb_W[0], comb_b[0], wvall,
                            True)

    bi0 = (4 * (argm[4] == 0).astype(jnp.int32)
           + 2 * (argm[0] == 1).astype(jnp.int32)
           + (argm[2] == 2).astype(jnp.int32))
    x1, hs1 = lax.switch(bi0, [
        l0_rw_1h_sum, l0_rw_1h_max, l0_rw_2h_sum, l0_rw_2h_max,
        l0_sym_1h_sum, l0_sym_1h_max, l0_sym_2h_sum, l0_sym_2h_max,
    ])


    def l1_1h_sum():
        parts = _sc_prop_sum(hs1, ei3, zrows)
        return _tc_comb1jk(parts, None, x1, h0, comb_W[1], comb_b[1],
                           jk_W, jk_b, wvall, False)

    def l1_1h_max():
        srcseg, dstseg, orows, z16 = cold_inputs()
        degp = _sc_degree(ei3, orows, z16)
        pm = _tc_segmax(hs1, srcseg, dstseg)
        return _tc_comb1jk(pm, degp, x1, h0, comb_W[1], comb_b[1],
                           jk_W, jk_b, wvall, True)

    def l1_2h_sum():
        p1 = _sc_prop_sum(hs1, ei3, zrows)
        hmid = _tc_mid(p1, None, wvall, 5, 9, False)
        p2 = _sc_prop_sum(hmid, ei3, zrows)
        return _tc_comb1jk(p2, None, x1, h0, comb_W[1], comb_b[1],
                           jk_W, jk_b, wvall, False)

    def l1_2h_max():
        srcseg, dstseg, orows, z16 = cold_inputs()
        degp = _sc_degree(ei3, orows, z16)
        pm1 = _tc_segmax(hs1, srcseg, dstseg)
        hmid = _tc_mid(pm1, degp, wvall, 5, 9, True)
        pm2 = _tc_segmax(hmid, srcseg, dstseg)
        return _tc_comb1jk(pm2, degp, x1, h0, comb_W[1], comb_b[1],
                           jk_W, jk_b, wvall, True)

    bi1 = (2 * (argm[1] == 1).astype(jnp.int32)
           + (argm[3] == 2).astype(jnp.int32))
    return lax.switch(bi1, [l1_1h_sum, l1_1h_max, l1_2h_sum, l1_2h_max])
